# baseline (speedup 1.0000x reference)
import jax
import jax.numpy as jnp
from jax import lax
from jax.experimental import pallas as pl
from jax.experimental.pallas import tpu as pltpu
from jax.experimental.pallas import tpu_sc as plsc

B = 128
N = 32768
S = 5
K = 64
NC = 2
NSUB = 16
NW = NC * NSUB
RPW = B // NW
L = 16
C = 8192
NCHUNK = N // C
NBINS = 2048
HSTRIDE = NBINS
SLOTS = 64
CREGION = L * SLOTS

HIMASK1 = -2097152
HIMASK2 = -1024

_i32 = jnp.int32


def _sc_body(x_hbm, zt_hbm, out_hbm, xbuf0, zbuf0, xbuf1, zbuf1, hist,
             sufbuf, candbuf, outtmp, sem0, sem1):
    cid = lax.axis_index("c")
    sid = lax.axis_index("s")
    wid = cid * NSUB + sid

    iota = lax.iota(_i32, L)
    ones_v = iota * 0 + 1
    zeros_v = iota * 0
    iota_slots = iota * SLOTS

    def zero_body(i, carry):
        plsc.store_scatter(hist, [i * L + iota], iota * 0)
        return carry

    lax.fori_loop(0, (S * NBINS) // L, zero_body, _i32(0))

    bufs = ((xbuf0, zbuf0), (xbuf1, zbuf1))
    sems = (sem0, sem1)

    def launch(b, ci, k):
        xb, zb = bufs[k]
        sem = sems[k]
        hs = [pltpu.async_copy(x_hbm.at[b, pl.ds(ci * C, C)], xb, sem)]
        for j in range(S):
            hs.append(pltpu.async_copy(zt_hbm.at[j, b, pl.ds(ci * C, C)],
                                       zb.at[pl.ds(j * C, C)], sem))
        return hs

    def run_stream(shift, nbins, himask, pref_sc, handles, compact, cnt_v0):
        unroll = 8

        def make_group_body(xb, zb):
            def group_body(g0, carry):
                cnts = carry
                idxs = []
                masks = []
                stores = []
                for u in range(unroll):
                    g = g0 * unroll + u
                    xv = xb[pl.ds(g * L, L)]
                    for j in range(S):
                        zv = zb[pl.ds(j * C + g * L, L)]
                        v = xv + zv
                        bits = plsc.bitcast(v, _i32)
                        key = bits ^ ((bits >> 31) | _i32(-2147483648))
                        bin_ = lax.shift_right_logical(
                            key, _i32(shift)) & _i32(nbins - 1)
                        idxs.append(bin_ | _i32(j * HSTRIDE))
                        if himask is not None:
                            m = (key & _i32(himask)) == pref_sc[j]
                            masks.append(m)
                            if compact:
                                cv = cnts[j]
                                cidx = (iota_slots + cv) | _i32(j * CREGION)
                                ms = m & (cv < SLOTS)
                                stores.append((cidx, key, ms))
                                cnts = (cnts[:j]
                                        + (cv + m.astype(_i32),)
                                        + cnts[j + 1:])
                for t in range(unroll * S):
                    if himask is None:
                        plsc.addupdate_scatter(hist, [idxs[t]], ones_v)
                    else:
                        plsc.addupdate_scatter(hist, [idxs[t]], ones_v,
                                               mask=masks[t])
                for cidx, key, ms in stores:
                    plsc.store_scatter(candbuf, [cidx], key, mask=ms)
                return cnts
            return group_body

        carry = tuple(cnt_v0) if compact else (zeros_v,)
        for ci in range(NCHUNK):
            cur = ci % 2
            if ci + 1 < NCHUNK:
                nxt_handles = launch(bq[0], ci + 1, (ci + 1) % 2)
            for h in handles:
                h.wait()
            carry = lax.fori_loop(0, C // L // unroll,
                                  make_group_body(*bufs[cur]), carry)
            if ci + 1 < NCHUNK:
                handles = nxt_handles
        return carry if compact else None

    def run_select(nv, shift, krems, prefs):
        def suf_body(i, accs):
            jv = (nv - 1) - i
            outs = []
            for s in range(S):
                base = s * HSTRIDE + jv * L + iota
                hv = plsc.load_gather(hist, [base])
                plsc.store_scatter(hist, [base], zeros_v)
                acc = accs[s] + hv
                plsc.store_scatter(sufbuf, [base], acc)
                outs.append(acc)
            return tuple(outs)

        lax.fori_loop(0, nv, suf_body, (zeros_v,) * S)

        def vec_total(s, j):
            return jnp.sum(
                plsc.load_gather(sufbuf, [s * HSTRIDE + j * L + iota]))

        def bs_body(i, lohis):
            out = []
            for s in range(S):
                lo, hi = lohis[2 * s], lohis[2 * s + 1]
                mid = (lo + hi + 1) >> 1
                pred = vec_total(s, mid) >= krems[s]
                out.append(jnp.where(pred, mid, lo))
                out.append(jnp.where(pred, hi, mid - 1))
            return tuple(out)

        lohis = lax.fori_loop(0, 7, bs_body, (_i32(0), _i32(nv - 1)) * S)

        nkrems = []
        nprefs = []
        for s in range(S):
            krem = krems[s]
            jstar = lohis[2 * s]
            jn = jnp.minimum(jstar + 1, _i32(nv - 1))
            sv_j = plsc.load_gather(sufbuf, [s * HSTRIDE + jstar * L + iota])
            sv_jn = plsc.load_gather(sufbuf, [s * HSTRIDE + jn * L + iota])
            sv_jn = jnp.where(jstar < nv - 1, sv_jn, zeros_v)
            above = jnp.sum(sv_jn)
            hv = sv_j - sv_jn
            revv = lax.rev(hv, (0,))
            cums = plsc.cumsum(revv) + above
            lt = cums < krem
            lane_r = jnp.sum(lt.astype(_i32))
            strict_above = above + jnp.sum(jnp.where(lt, revv, zeros_v))
            binv = jstar * L + (L - 1) - lane_r
            nkrems.append(krem - strict_above)
            nprefs.append(prefs[s] | (binv << shift))
        return nkrems, nprefs

    bq = [None]

    def row_body(r, outacc):
        b = wid * RPW + r
        bq[0] = b

        handles = launch(b, 0, 0)
        run_stream(21, 2048, None, None, handles, False, None)
        handles = launch(b, 0, 0)
        krems, prefs = run_select(128, 21, [_i32(K)] * S, [_i32(0)] * S)

        cnts = run_stream(10, 2048, HIMASK1, prefs, handles, True,
                          (zeros_v,) * S)
        krems, prefs = run_select(128, 10, krems, prefs)

        maxs = [jnp.max(cnts[s]) for s in range(S)]
        ok = maxs[0] <= SLOTS
        mx = maxs[0]
        for s in range(1, S):
            ok = ok & (maxs[s] <= SLOTS)
            mx = jnp.maximum(mx, maxs[s])

        def mini_branch():
            nslots = jnp.minimum(mx, _i32(SLOTS))

            def mb(i, carry):
                for s in range(S):
                    kv = plsc.load_gather(
                        candbuf, [s * CREGION + iota_slots + i])
                    m = (i < cnts[s]) & ((kv & _i32(HIMASK2)) == prefs[s])
                    bin_ = (kv & _i32(1023)) | _i32(s * HSTRIDE)
                    plsc.addupdate_scatter(hist, [bin_], ones_v, mask=m)
                return carry

            lax.fori_loop(0, nslots, mb, _i32(0))
            _, pf = run_select(64, 0, krems, prefs)
            return tuple(pf)

        def full_branch():
            hs = launch(b, 0, 0)
            run_stream(0, 1024, HIMASK2, prefs, hs, False, None)
            _, pf = run_select(64, 0, krems, prefs)
            return tuple(pf)

        prefs_f = lax.cond(ok, mini_branch, full_branch)

        keysv = zeros_v
        for s in range(S):
            keysv = jnp.where(iota == s, prefs_f[s], keysv)
        xm = jnp.where(keysv < 0, _i32(-2147483648), _i32(-1))
        vals = plsc.bitcast(keysv ^ xm, jnp.float32)
        vals = jnp.where(iota < S, vals, jnp.float32(0.0))
        mean = jnp.sum(vals) * jnp.float32(1.0 / S)
        return jnp.where(iota == r, mean, outacc)

    outacc = lax.fori_loop(0, RPW, row_body, jnp.zeros((L,), jnp.float32))
    outtmp[...] = outacc
    pltpu.sync_copy(outtmp, out_hbm.at[pl.ds(wid * L, L)])


def kernel(x, Z):
    mesh = plsc.VectorSubcoreMesh(core_axis_name="c", subcore_axis_name="s",
                                  num_cores=NC, num_subcores=NSUB)
    sck = pl.kernel(
        _sc_body,
        out_type=jax.ShapeDtypeStruct((NW * L,), jnp.float32),
        mesh=mesh,
        compiler_params=pltpu.CompilerParams(needs_layout_passes=False),
        scratch_types=[
            pltpu.VMEM((C,), jnp.float32),
            pltpu.VMEM((S * C,), jnp.float32),
            pltpu.VMEM((C,), jnp.float32),
            pltpu.VMEM((S * C,), jnp.float32),
            pltpu.VMEM((S * NBINS,), _i32),
            pltpu.VMEM((S * NBINS,), _i32),
            pltpu.VMEM((S * CREGION,), _i32),
            pltpu.VMEM((L,), jnp.float32),
            pltpu.SemaphoreType.DMA,
            pltpu.SemaphoreType.DMA,
        ],
    )
    out = sck(x, jnp.transpose(Z, (2, 0, 1)))
    return out.reshape(NW, L)[:, :RPW].reshape(B)


# speedup vs baseline: 1.4035x; 1.4035x over previous
import jax
import jax.numpy as jnp
from jax import lax
from jax.experimental import pallas as pl
from jax.experimental.pallas import tpu as pltpu
from jax.experimental.pallas import tpu_sc as plsc

B = 128
N = 32768
S = 5
K = 64
NC = 2
NSUB = 16
NW = NC * NSUB
RPW = B // NW
L = 16
C = 8192
NCHUNK = N // C
NBINS = 2048
HSTRIDE = NBINS

PASSES = ((21, 2048, 0), (10, 2048, -2097152), (0, 1024, -1024))

_i32 = jnp.int32


def _sc_body(x_hbm, zt_hbm, out_hbm, xbuf0, zbuf0, xbuf1, zbuf1, hist,
             sufbuf, outtmp, sem0, sem1):
    cid = lax.axis_index("c")
    sid = lax.axis_index("s")
    wid = cid * NSUB + sid

    iota = lax.iota(_i32, L)
    ones_v = iota * 0 + 1
    zeros_v = iota * 0

    def zero_body(i, carry):
        plsc.store_scatter(hist, [i * L + iota], iota * 0)
        return carry

    lax.fori_loop(0, (S * NBINS) // L, zero_body, _i32(0))

    bufs = ((xbuf0, zbuf0), (xbuf1, zbuf1))
    sems = (sem0, sem1)

    def launch(b, ci, k):
        xb, zb = bufs[k]
        sem = sems[k]
        hs = [pltpu.async_copy(x_hbm.at[b, pl.ds(ci * C, C)], xb, sem)]
        for j in range(S):
            hs.append(pltpu.async_copy(zt_hbm.at[j, b, pl.ds(ci * C, C)],
                                       zb.at[pl.ds(j * C, C)], sem))
        return hs

    def row_body(r, outacc):
        b = wid * RPW + r

        prefs = [_i32(0)] * S
        krems = [_i32(K)] * S

        for p, (shift, nbins, himask) in enumerate(PASSES):
            nv = nbins // L

            pref_sc = list(prefs)

            if p == 0:
                handles = launch(b, 0, 0)

            def make_group_body(xb, zb):
                def group_body(g0, gcarry):
                    idxs = []
                    masks = []
                    for u in range(8):
                        g = g0 * 8 + u
                        xv = xb[pl.ds(g * L, L)]
                        for j in range(S):
                            zv = zb[pl.ds(j * C + g * L, L)]
                            v = xv + zv
                            bits = plsc.bitcast(v, _i32)
                            key = bits ^ ((bits >> 31) | _i32(-2147483648))
                            bin_ = lax.shift_right_logical(
                                key, _i32(shift)) & _i32(nbins - 1)
                            idxs.append(bin_ | _i32(j * HSTRIDE))
                            if p != 0:
                                masks.append(
                                    (key & _i32(himask)) == pref_sc[j])
                    for t in range(8 * S):
                        if p == 0:
                            plsc.addupdate_scatter(hist, [idxs[t]], ones_v)
                        else:
                            plsc.addupdate_scatter(hist, [idxs[t]], ones_v,
                                                   mask=masks[t])
                    return gcarry
                return group_body

            for ci in range(NCHUNK):
                cur = ci % 2
                if ci + 1 < NCHUNK:
                    nxt_handles = launch(b, ci + 1, (ci + 1) % 2)
                for h in handles:
                    h.wait()
                lax.fori_loop(0, C // L // 8,
                              make_group_body(*bufs[cur]), _i32(0))
                if ci + 1 < NCHUNK:
                    handles = nxt_handles
            if p < len(PASSES) - 1:
                handles = launch(b, 0, 0)


            def suf_body(i, accs):
                jv = (nv - 1) - i
                outs = []
                for s in range(S):
                    base = s * HSTRIDE + jv * L + iota
                    hv = plsc.load_gather(hist, [base])
                    plsc.store_scatter(hist, [base], zeros_v)
                    acc = accs[s] + hv
                    plsc.store_scatter(sufbuf, [base], acc)
                    outs.append(acc)
                return tuple(outs)

            lax.fori_loop(0, nv, suf_body, (zeros_v,) * S)

            def vec_total(s, j):
                return jnp.sum(
                    plsc.load_gather(sufbuf, [s * HSTRIDE + j * L + iota]))

            def bs_body(i, lohis):
                out = []
                for s in range(S):
                    lo, hi = lohis[2 * s], lohis[2 * s + 1]
                    mid = (lo + hi + 1) >> 1
                    pred = vec_total(s, mid) >= krems[s]
                    out.append(jnp.where(pred, mid, lo))
                    out.append(jnp.where(pred, hi, mid - 1))
                return tuple(out)

            lohis = lax.fori_loop(0, 7, bs_body,
                                  (_i32(0), _i32(nv - 1)) * S)

            for s in range(S):
                krem = krems[s]
                jstar = lohis[2 * s]
                jn = jnp.minimum(jstar + 1, _i32(nv - 1))
                sv_j = plsc.load_gather(sufbuf,
                                        [s * HSTRIDE + jstar * L + iota])
                sv_jn = plsc.load_gather(sufbuf,
                                         [s * HSTRIDE + jn * L + iota])
                sv_jn = jnp.where(jstar < nv - 1, sv_jn, zeros_v)
                above = jnp.sum(sv_jn)
                hv = sv_j - sv_jn
                revv = lax.rev(hv, (0,))
                cums = plsc.cumsum(revv) + above
                lt = cums < krem
                lane_r = jnp.sum(lt.astype(_i32))
                strict_above = above + jnp.sum(jnp.where(lt, revv, zeros_v))
                binv = jstar * L + (L - 1) - lane_r

                krems[s] = krem - strict_above
                prefs[s] = prefs[s] | (binv << shift)

        keysv = zeros_v
        for s in range(S):
            keysv = jnp.where(iota == s, prefs[s], keysv)
        xm = jnp.where(keysv < 0, _i32(-2147483648), _i32(-1))
        vals = plsc.bitcast(keysv ^ xm, jnp.float32)
        vals = jnp.where(iota < S, vals, jnp.float32(0.0))
        mean = jnp.sum(vals) * jnp.float32(1.0 / S)
        return jnp.where(iota == r, mean, outacc)

    outacc = lax.fori_loop(0, RPW, row_body, jnp.zeros((L,), jnp.float32))
    outtmp[...] = outacc
    pltpu.sync_copy(outtmp, out_hbm.at[pl.ds(wid * L, L)])


def kernel(x, Z):
    mesh = plsc.VectorSubcoreMesh(core_axis_name="c", subcore_axis_name="s",
                                  num_cores=NC, num_subcores=NSUB)
    sck = pl.kernel(
        _sc_body,
        out_type=jax.ShapeDtypeStruct((NW * L,), jnp.float32),
        mesh=mesh,
        compiler_params=pltpu.CompilerParams(needs_layout_passes=False),
        scratch_types=[
            pltpu.VMEM((C,), jnp.float32),
            pltpu.VMEM((S * C,), jnp.float32),
            pltpu.VMEM((C,), jnp.float32),
            pltpu.VMEM((S * C,), jnp.float32),
            pltpu.VMEM((S * NBINS,), _i32),
            pltpu.VMEM((S * NBINS,), _i32),
            pltpu.VMEM((L,), jnp.float32),
            pltpu.SemaphoreType.DMA,
            pltpu.SemaphoreType.DMA,
        ],
    )
    out = sck(x, jnp.transpose(Z, (2, 0, 1)))
    return out.reshape(NW, L)[:, :RPW].reshape(B)
